# baseline (speedup 1.0000x reference)
import functools
import jax
import jax.numpy as jnp
from jax import lax
from jax.experimental import pallas as pl
from jax.experimental.pallas import tpu as pltpu
from jax.experimental.pallas import tpu_sc as plsc

B = 32
N = 1024
KNB = 16
BN = B * N
DPAD = 128

_F32 = jnp.float32
_BF16 = jnp.bfloat16
_BIG = 3.0e38


def _knn_body(q_ref, c_ref, idx_ref, *, rb, boff):
    q = q_ref[0]
    c = c_ref[0]
    csq = jnp.sum(c * c, axis=1, keepdims=True)
    dn = (((1,), (1,)), ((), ()))
    g = lax.dot_general(q.astype(_BF16), c.astype(_BF16), dn,
                        preferred_element_type=_F32)
    ones = jnp.ones((rb, 1), _F32)
    ccb = lax.dot_general(ones, csq, dn, precision=lax.Precision.HIGHEST,
                          preferred_element_type=_F32)
    work = ccb - 2.0 * g
    colid = lax.broadcasted_iota(jnp.int32, (rb, N), 1).astype(_F32)
    base = (pl.program_id(0) + boff) * N
    cols = []
    for _ in range(KNB):
        m = jnp.min(work, axis=1, keepdims=True)
        cand = jnp.where(work == m, colid, 1.0e9)
        a = jnp.min(cand, axis=1, keepdims=True)
        cols.append(a)
        work = jnp.where(cand == a, _BIG, work)
    idx_ref[0] = jnp.concatenate(cols, axis=1).astype(jnp.int32) + base


def _knn_topk(h3, boff, nb):
    d = h3.shape[2]
    rb = 256
    grid = (nb, N // rb)
    return pl.pallas_call(
        functools.partial(_knn_body, rb=rb, boff=boff),
        grid=grid,
        in_specs=[
            pl.BlockSpec((1, rb, d), lambda b, r: (b + boff, r, 0)),
            pl.BlockSpec((1, N, d), lambda b, r: (b + boff, 0, 0)),
        ],
        out_specs=pl.BlockSpec((1, rb, KNB), lambda b, r: (b, r, 0)),
        out_shape=jax.ShapeDtypeStruct((nb, N, KNB), jnp.int32),
    )(h3, h3)


_GC = 128


def _sc_gather_body(n_edges, tab_hbm, idx_hbm, out_hbm,
                    idx_a, idx_b, rows_a, rows_b, sem_a, sem_b):
    nc = 2
    wid = lax.axis_index("s") * nc + lax.axis_index("c")
    per_w = n_edges // 32
    base = wid * per_w
    nch = per_w // _GC
    slots = ((idx_a, rows_a, sem_a), (idx_b, rows_b, sem_b))

    pltpu.sync_copy(idx_hbm.at[pl.ds(base, _GC)], idx_a)
    pltpu.async_copy(tab_hbm.at[idx_a], rows_a, sem_a)

    def body(j, carry):
        for b in (0, 1):
            i = 2 * j + b
            idx_c, rows_c, sem_c = slots[b]
            idx_n, rows_n, sem_n = slots[1 - b]

            @pl.when(i + 1 < nch)
            def _():
                eb_n = base + (i + 1) * _GC
                pltpu.sync_copy(idx_hbm.at[pl.ds(eb_n, _GC)], idx_n)
                pltpu.async_copy(tab_hbm.at[idx_n], rows_n, sem_n)

            pltpu.make_async_copy(tab_hbm.at[idx_c], rows_c, sem_c).wait()
            pltpu.sync_copy(rows_c, out_hbm.at[pl.ds(base + i * _GC, _GC)])
        return carry

    lax.fori_loop(0, nch // 2, body, 0)


@functools.lru_cache(maxsize=None)
def _make_sc_gather(n_edges):
    mesh = plsc.VectorSubcoreMesh(core_axis_name="c", subcore_axis_name="s")
    return pl.kernel(
        functools.partial(_sc_gather_body, n_edges),
        out_type=jax.ShapeDtypeStruct((n_edges, DPAD), _F32),
        mesh=mesh,
        scratch_types=[
            pltpu.VMEM((_GC,), jnp.int32),
            pltpu.VMEM((_GC,), jnp.int32),
            pltpu.VMEM((_GC, DPAD), _F32),
            pltpu.VMEM((_GC, DPAD), _F32),
            pltpu.SemaphoreType.DMA,
            pltpu.SemaphoreType.DMA,
        ],
    )


def _sc_gather(table, idx_flat):
    return _make_sc_gather(idx_flat.shape[0])(table, idx_flat)


def _edgeconv_body(xj_ref, h_ref, th_ref, ph_ref, o_ref):
    xi = h_ref[...]
    thb = th_ref[...].astype(_BF16)
    phb = ph_ref[...].astype(_BF16)
    dn = (((1,), (1,)), ((), ()))
    p = lax.dot_general(xi.astype(_BF16), phb, dn, preferred_element_type=_F32)
    m = None
    for k in range(KNB):
        d = (xj_ref[k] - xi).astype(_BF16)
        mk = lax.dot_general(d, thb, dn, preferred_element_type=_F32)
        m = mk if m is None else jnp.maximum(m, mk)
    out = m + p
    o_ref[...] = jnp.where(out >= 0.0, out, 0.2 * out)


def _edgeconv(xj, h_pad, th_pad, ph_pad, roff, nrows):
    dout_pad = th_pad.shape[0]
    rb = 256
    grid = (nrows // rb,)
    ro = roff // rb
    xj3 = xj.reshape(KNB, nrows, DPAD)
    return pl.pallas_call(
        _edgeconv_body,
        grid=grid,
        in_specs=[
            pl.BlockSpec((KNB, rb, DPAD), lambda i: (0, i, 0)),
            pl.BlockSpec((rb, DPAD), lambda i: (i + ro, 0)),
            pl.BlockSpec((dout_pad, DPAD), lambda i: (0, 0)),
            pl.BlockSpec((dout_pad, DPAD), lambda i: (0, 0)),
        ],
        out_specs=pl.BlockSpec((rb, dout_pad), lambda i: (i, 0)),
        out_shape=jax.ShapeDtypeStruct((nrows, dout_pad), _F32),
    )(xj3, h_pad, th_pad, ph_pad)


def _proj_pool_body(h1_ref, h2_ref, h3_ref, h4_ref, w_ref, b_ref,
                    pmax_ref, pavg_ref):
    hcat = jnp.concatenate(
        [h1_ref[0][:, :64], h2_ref[0][:, :64], h3_ref[0], h4_ref[0]],
        axis=1)
    dn = (((1,), (1,)), ((), ()))
    pr = lax.dot_general(hcat.astype(_BF16), w_ref[...].astype(_BF16), dn,
                         preferred_element_type=_F32)
    pr = pr + b_ref[...]
    pmax_ref[0] = jnp.max(pr, axis=0, keepdims=True)
    pavg_ref[0] = jnp.sum(pr, axis=0, keepdims=True) * (1.0 / N)


def _proj_pool(hs, proj_w, proj_b):
    dproj, dcat = proj_w.shape
    nb = hs[0].shape[0]
    dims = [h.shape[2] for h in hs]
    specs = [pl.BlockSpec((1, N, d), lambda b: (b, 0, 0)) for d in dims]
    return pl.pallas_call(
        _proj_pool_body,
        grid=(nb,),
        in_specs=specs + [
            pl.BlockSpec((dproj, dcat), lambda b: (0, 0)),
            pl.BlockSpec((1, dproj), lambda b: (0, 0)),
        ],
        out_specs=[
            pl.BlockSpec((1, 1, dproj), lambda b: (b, 0, 0)),
            pl.BlockSpec((1, 1, dproj), lambda b: (b, 0, 0)),
        ],
        out_shape=[
            jax.ShapeDtypeStruct((nb, 1, dproj), _F32),
            jax.ShapeDtypeStruct((nb, 1, dproj), _F32),
        ],
    )(*hs, proj_w, proj_b.reshape(1, dproj))


def _mlp_body(pmax_ref, pavg_ref, w0_ref, b0_ref, g0_ref, bb0_ref,
              w1_ref, b1_ref, g1_ref, bb1_ref, wo_ref, bo_ref, out_ref):
    h = jnp.concatenate([pmax_ref[...], pavg_ref[...]], axis=1)
    dn = (((1,), (1,)), ((), ()))

    def block(h, w_ref, b_ref, g_ref, bb_ref):
        h = lax.dot_general(h.astype(_BF16), w_ref[...].astype(_BF16), dn,
                            preferred_element_type=_F32)
        h = h + b_ref[...]
        mean = jnp.sum(h, axis=0, keepdims=True) * (1.0 / B)
        d = h - mean
        var = jnp.sum(d * d, axis=0, keepdims=True) * (1.0 / B)
        h = d / jnp.sqrt(var + 1e-5) * g_ref[...] + bb_ref[...]
        return jnp.where(h >= 0.0, h, 0.2 * h)

    h = block(h, w0_ref, b0_ref, g0_ref, bb0_ref)
    h = block(h, w1_ref, b1_ref, g1_ref, bb1_ref)
    out = lax.dot_general(h.astype(_BF16), wo_ref[...].astype(_BF16), dn,
                          preferred_element_type=_F32)
    out_ref[...] = out + bo_ref[...]


def _mlp_head(pmax, pavg, emb_w_0, emb_b_0, bn_g_0, bn_b_0,
              emb_w_1, emb_b_1, bn_g_1, bn_b_1, out_w, out_b):
    args = [
        pmax, pavg,
        emb_w_0, emb_b_0.reshape(1, -1), bn_g_0.reshape(1, -1),
        bn_b_0.reshape(1, -1),
        emb_w_1, emb_b_1.reshape(1, -1), bn_g_1.reshape(1, -1),
        bn_b_1.reshape(1, -1),
        out_w, out_b.reshape(1, -1),
    ]
    nclass = out_w.shape[0]
    return pl.pallas_call(
        _mlp_body,
        out_shape=jax.ShapeDtypeStruct((B, nclass), _F32),
    )(*args)


def _pad_w(w):
    dout, din = w.shape
    dout_pad = 128 if dout < 128 else dout
    return jnp.pad(w, ((0, dout_pad - dout), (0, DPAD - din)))


@jax.jit
def kernel(x, theta_0, phi_0, theta_1, phi_1, theta_2, phi_2, theta_3, phi_3,
           proj_w, proj_b, emb_w_0, emb_b_0, bn_g_0, bn_b_0,
           emb_w_1, emb_b_1, bn_g_1, bn_b_1, out_w, out_b):
    thetas = [_pad_w(w) for w in (theta_0, theta_1, theta_2, theta_3)]
    phis = [_pad_w(w) for w in (phi_0, phi_1, phi_2, phi_3)]
    h_pad = jnp.pad(x.reshape(BN, 3), ((0, 0), (0, DPAD - 3)))
    hb = B // 2
    hr = hb * N
    halves = [h_pad[:hr], h_pad[hr:]]
    hs = [[], []]
    for li in range(4):
        houts = []
        for s in range(2):
            hp = halves[s]
            h3 = hp.reshape(hb, N, DPAD)
            idx = _knn_topk(h3, 0, hb)
            xj = _sc_gather(hp, idx.transpose(2, 0, 1).reshape(-1))
            houts.append((hp, idx, xj))
        for s in range(2):
            hp, idx, xj = houts[s]
            ho = _edgeconv(xj, hp, thetas[li], phis[li], 0, hr)
            hs[s].append(ho.reshape(hb, N, -1))
            if li < 3:
                halves[s] = ho
    pools = [_proj_pool(hs[s], proj_w, proj_b) for s in range(2)]
    pmax = jnp.concatenate([p[0] for p in pools], axis=0).reshape(B, -1)
    pavg = jnp.concatenate([p[1] for p in pools], axis=0).reshape(B, -1)
    return _mlp_head(pmax, pavg, emb_w_0, emb_b_0, bn_g_0, bn_b_0,
                     emb_w_1, emb_b_1, bn_g_1, bn_b_1, out_w, out_b)


# speedup vs baseline: 1.0469x; 1.0469x over previous
import functools
import jax
import jax.numpy as jnp
from jax import lax
from jax.experimental import pallas as pl
from jax.experimental.pallas import tpu as pltpu
from jax.experimental.pallas import tpu_sc as plsc

B = 32
N = 1024
KNB = 16
BN = B * N
DPAD = 128

_F32 = jnp.float32
_BF16 = jnp.bfloat16
_BIG = 3.0e38


def _knn_body(q_ref, c_ref, idx_ref, *, rb, boff):
    q = q_ref[0]
    c = c_ref[0]
    csq = jnp.sum(c * c, axis=1, keepdims=True)
    dn = (((1,), (1,)), ((), ()))
    g = lax.dot_general(q.astype(_BF16), c.astype(_BF16), dn,
                        preferred_element_type=_F32)
    ones = jnp.ones((rb, 1), _F32)
    ccb = lax.dot_general(ones, csq, dn, precision=lax.Precision.HIGHEST,
                          preferred_element_type=_F32)
    work = ccb - 2.0 * g
    colid = lax.broadcasted_iota(jnp.int32, (rb, N), 1).astype(_F32)
    base = (pl.program_id(0) + boff) * N
    cols = []
    for _ in range(KNB):
        m = jnp.min(work, axis=1, keepdims=True)
        cand = jnp.where(work == m, colid, 1.0e9)
        a = jnp.min(cand, axis=1, keepdims=True)
        cols.append(a)
        work = jnp.where(cand == a, _BIG, work)
    idx_ref[0] = jnp.concatenate(cols, axis=1).astype(jnp.int32) + base


def _knn_topk(h3, boff, nb):
    d = h3.shape[2]
    rb = 256
    grid = (nb, N // rb)
    return pl.pallas_call(
        functools.partial(_knn_body, rb=rb, boff=boff),
        grid=grid,
        in_specs=[
            pl.BlockSpec((1, rb, d), lambda b, r: (b + boff, r, 0)),
            pl.BlockSpec((1, N, d), lambda b, r: (b + boff, 0, 0)),
        ],
        out_specs=pl.BlockSpec((1, rb, KNB), lambda b, r: (b, r, 0)),
        out_shape=jax.ShapeDtypeStruct((nb, N, KNB), jnp.int32),
    )(h3, h3)


_GC = 128


def _sc_gather_body(n_edges, tab_hbm, idx_hbm, out_hbm,
                    idx_a, idx_b, rows_a, rows_b, sem_a, sem_b):
    nc = 2
    wid = lax.axis_index("s") * nc + lax.axis_index("c")
    per_w = n_edges // 32
    base = wid * per_w
    nch = per_w // _GC
    slots = ((idx_a, rows_a, sem_a), (idx_b, rows_b, sem_b))

    pltpu.sync_copy(idx_hbm.at[pl.ds(base, _GC)], idx_a)
    pltpu.async_copy(tab_hbm.at[idx_a], rows_a, sem_a)

    def body(j, carry):
        for b in (0, 1):
            i = 2 * j + b
            idx_c, rows_c, sem_c = slots[b]
            idx_n, rows_n, sem_n = slots[1 - b]

            @pl.when(i + 1 < nch)
            def _():
                eb_n = base + (i + 1) * _GC
                pltpu.sync_copy(idx_hbm.at[pl.ds(eb_n, _GC)], idx_n)
                pltpu.async_copy(tab_hbm.at[idx_n], rows_n, sem_n)

            pltpu.make_async_copy(tab_hbm.at[idx_c], rows_c, sem_c).wait()
            pltpu.sync_copy(rows_c, out_hbm.at[pl.ds(base + i * _GC, _GC)])
        return carry

    lax.fori_loop(0, nch // 2, body, 0)


@functools.lru_cache(maxsize=None)
def _make_sc_gather(n_edges):
    mesh = plsc.VectorSubcoreMesh(core_axis_name="c", subcore_axis_name="s")
    return pl.kernel(
        functools.partial(_sc_gather_body, n_edges),
        out_type=jax.ShapeDtypeStruct((n_edges, DPAD), _F32),
        mesh=mesh,
        scratch_types=[
            pltpu.VMEM((_GC,), jnp.int32),
            pltpu.VMEM((_GC,), jnp.int32),
            pltpu.VMEM((_GC, DPAD), _F32),
            pltpu.VMEM((_GC, DPAD), _F32),
            pltpu.SemaphoreType.DMA,
            pltpu.SemaphoreType.DMA,
        ],
    )


def _sc_gather(table, idx_flat):
    return _make_sc_gather(idx_flat.shape[0])(table, idx_flat)


def _edgeconv_body(xj_ref, h_ref, th_ref, ph_ref, o_ref):
    xi = h_ref[...]
    thb = th_ref[...].astype(_BF16)
    phb = ph_ref[...].astype(_BF16)
    dn = (((1,), (1,)), ((), ()))
    p = lax.dot_general(xi.astype(_BF16), phb, dn, preferred_element_type=_F32)
    m = None
    for k in range(KNB):
        d = (xj_ref[k] - xi).astype(_BF16)
        mk = lax.dot_general(d, thb, dn, preferred_element_type=_F32)
        m = mk if m is None else jnp.maximum(m, mk)
    out = m + p
    o_ref[...] = jnp.where(out >= 0.0, out, 0.2 * out)


def _edgeconv(xj, h_pad, th_pad, ph_pad, roff, nrows):
    dout_pad = th_pad.shape[0]
    rb = 256
    grid = (nrows // rb,)
    ro = roff // rb
    xj3 = xj.reshape(KNB, nrows, DPAD)
    return pl.pallas_call(
        _edgeconv_body,
        grid=grid,
        in_specs=[
            pl.BlockSpec((KNB, rb, DPAD), lambda i: (0, i, 0)),
            pl.BlockSpec((rb, DPAD), lambda i: (i + ro, 0)),
            pl.BlockSpec((dout_pad, DPAD), lambda i: (0, 0)),
            pl.BlockSpec((dout_pad, DPAD), lambda i: (0, 0)),
        ],
        out_specs=pl.BlockSpec((rb, dout_pad), lambda i: (i, 0)),
        out_shape=jax.ShapeDtypeStruct((nrows, dout_pad), _F32),
    )(xj3, h_pad, th_pad, ph_pad)


def _proj_pool_body(h1_ref, h2_ref, h3_ref, h4_ref, w_ref, b_ref,
                    pmax_ref, pavg_ref):
    hcat = jnp.concatenate(
        [h1_ref[0][:, :64], h2_ref[0][:, :64], h3_ref[0], h4_ref[0]],
        axis=1)
    dn = (((1,), (1,)), ((), ()))
    pr = lax.dot_general(hcat.astype(_BF16), w_ref[...].astype(_BF16), dn,
                         preferred_element_type=_F32)
    pr = pr + b_ref[...]
    pmax_ref[0] = jnp.max(pr, axis=0, keepdims=True)
    pavg_ref[0] = jnp.sum(pr, axis=0, keepdims=True) * (1.0 / N)


def _proj_pool(hs, proj_w, proj_b):
    dproj, dcat = proj_w.shape
    nb = hs[0].shape[0]
    dims = [h.shape[2] for h in hs]
    specs = [pl.BlockSpec((1, N, d), lambda b: (b, 0, 0)) for d in dims]
    return pl.pallas_call(
        _proj_pool_body,
        grid=(nb,),
        in_specs=specs + [
            pl.BlockSpec((dproj, dcat), lambda b: (0, 0)),
            pl.BlockSpec((1, dproj), lambda b: (0, 0)),
        ],
        out_specs=[
            pl.BlockSpec((1, 1, dproj), lambda b: (b, 0, 0)),
            pl.BlockSpec((1, 1, dproj), lambda b: (b, 0, 0)),
        ],
        out_shape=[
            jax.ShapeDtypeStruct((nb, 1, dproj), _F32),
            jax.ShapeDtypeStruct((nb, 1, dproj), _F32),
        ],
    )(*hs, proj_w, proj_b.reshape(1, dproj))


def _mlp_body(pmax_ref, pavg_ref, w0_ref, b0_ref, g0_ref, bb0_ref,
              w1_ref, b1_ref, g1_ref, bb1_ref, wo_ref, bo_ref, out_ref):
    h = jnp.concatenate([pmax_ref[...], pavg_ref[...]], axis=1)
    dn = (((1,), (1,)), ((), ()))

    def block(h, w_ref, b_ref, g_ref, bb_ref):
        h = lax.dot_general(h.astype(_BF16), w_ref[...].astype(_BF16), dn,
                            preferred_element_type=_F32)
        h = h + b_ref[...]
        mean = jnp.sum(h, axis=0, keepdims=True) * (1.0 / B)
        d = h - mean
        var = jnp.sum(d * d, axis=0, keepdims=True) * (1.0 / B)
        h = d / jnp.sqrt(var + 1e-5) * g_ref[...] + bb_ref[...]
        return jnp.where(h >= 0.0, h, 0.2 * h)

    h = block(h, w0_ref, b0_ref, g0_ref, bb0_ref)
    h = block(h, w1_ref, b1_ref, g1_ref, bb1_ref)
    out = lax.dot_general(h.astype(_BF16), wo_ref[...].astype(_BF16), dn,
                          preferred_element_type=_F32)
    out_ref[...] = out + bo_ref[...]


def _mlp_head(pmax, pavg, emb_w_0, emb_b_0, bn_g_0, bn_b_0,
              emb_w_1, emb_b_1, bn_g_1, bn_b_1, out_w, out_b):
    args = [
        pmax, pavg,
        emb_w_0, emb_b_0.reshape(1, -1), bn_g_0.reshape(1, -1),
        bn_b_0.reshape(1, -1),
        emb_w_1, emb_b_1.reshape(1, -1), bn_g_1.reshape(1, -1),
        bn_b_1.reshape(1, -1),
        out_w, out_b.reshape(1, -1),
    ]
    nclass = out_w.shape[0]
    return pl.pallas_call(
        _mlp_body,
        out_shape=jax.ShapeDtypeStruct((B, nclass), _F32),
    )(*args)


def _pad_w(w):
    dout, din = w.shape
    dout_pad = 128 if dout < 128 else dout
    return jnp.pad(w, ((0, dout_pad - dout), (0, DPAD - din)))


@jax.jit
def kernel(x, theta_0, phi_0, theta_1, phi_1, theta_2, phi_2, theta_3, phi_3,
           proj_w, proj_b, emb_w_0, emb_b_0, bn_g_0, bn_b_0,
           emb_w_1, emb_b_1, bn_g_1, bn_b_1, out_w, out_b):
    thetas = [_pad_w(w) for w in (theta_0, theta_1, theta_2, theta_3)]
    phis = [_pad_w(w) for w in (phi_0, phi_1, phi_2, phi_3)]
    h_pad = jnp.pad(x.reshape(BN, 3), ((0, 0), (0, DPAD - 3)))
    hs = []
    hb = B // 2
    for li in range(4):
        h3 = h_pad.reshape(B, N, DPAD)
        idx0 = _knn_topk(h3, 0, hb)
        xj0 = _sc_gather(h_pad, idx0.transpose(2, 0, 1).reshape(-1))
        idx1 = _knn_topk(h3, hb, hb)
        xj1 = _sc_gather(h_pad, idx1.transpose(2, 0, 1).reshape(-1))
        ho0 = _edgeconv(xj0, h_pad, thetas[li], phis[li], 0, hb * N)
        ho1 = _edgeconv(xj1, h_pad, thetas[li], phis[li], hb * N, hb * N)
        h_out = jnp.concatenate([ho0, ho1], axis=0)
        hs.append(h_out.reshape(B, N, -1))
        if li < 3:
            h_pad = h_out[:, :DPAD] if h_out.shape[1] > DPAD else h_out
    pmax, pavg = _proj_pool(hs, proj_w, proj_b)
    pmax = pmax.reshape(B, -1)
    pavg = pavg.reshape(B, -1)
    return _mlp_head(pmax, pavg, emb_w_0, emb_b_0, bn_g_0, bn_b_0,
                     emb_w_1, emb_b_1, bn_g_1, bn_b_1, out_w, out_b)


# speedup vs baseline: 1.0640x; 1.0163x over previous
import functools
import jax
import jax.numpy as jnp
from jax import lax
from jax.experimental import pallas as pl
from jax.experimental.pallas import tpu as pltpu
from jax.experimental.pallas import tpu_sc as plsc

B = 32
N = 1024
KNB = 16
BN = B * N
DPAD = 128

_F32 = jnp.float32
_BF16 = jnp.bfloat16
_BIG = 3.0e38


def _knn_body(q_ref, c_ref, idx_ref, *, rb, boff):
    q = q_ref[0]
    c = c_ref[0]
    csq = jnp.sum(c * c, axis=1, keepdims=True)
    dn = (((1,), (1,)), ((), ()))
    g = lax.dot_general(q.astype(_BF16), c.astype(_BF16), dn,
                        preferred_element_type=_F32)
    ones = jnp.ones((rb, 1), _F32)
    ccb = lax.dot_general(ones, csq, dn, precision=lax.Precision.HIGHEST,
                          preferred_element_type=_F32)
    work = ccb - 2.0 * g
    colid = lax.broadcasted_iota(jnp.int32, (rb, N), 1).astype(_F32)
    base = (pl.program_id(0) + boff) * N
    cols = []
    for _ in range(KNB):
        m = jnp.min(work, axis=1, keepdims=True)
        cand = jnp.where(work == m, colid, 1.0e9)
        a = jnp.min(cand, axis=1, keepdims=True)
        cols.append(a)
        work = jnp.where(cand == a, _BIG, work)
    idx_ref[0] = jnp.concatenate(cols, axis=1).astype(jnp.int32) + base


def _knn_topk(h3, boff, nb):
    d = h3.shape[2]
    rb = 512
    grid = (nb, N // rb)
    return pl.pallas_call(
        functools.partial(_knn_body, rb=rb, boff=boff),
        grid=grid,
        in_specs=[
            pl.BlockSpec((1, rb, d), lambda b, r: (b + boff, r, 0)),
            pl.BlockSpec((1, N, d), lambda b, r: (b + boff, 0, 0)),
        ],
        out_specs=pl.BlockSpec((1, rb, KNB), lambda b, r: (b, r, 0)),
        out_shape=jax.ShapeDtypeStruct((nb, N, KNB), jnp.int32),
    )(h3, h3)


_GC = 128


def _sc_gather_body(n_edges, tab_hbm, idx_hbm, out_hbm,
                    idx_a, idx_b, rows_a, rows_b, sem_a, sem_b):
    nc = 2
    wid = lax.axis_index("s") * nc + lax.axis_index("c")
    per_w = n_edges // 32
    base = wid * per_w
    nch = per_w // _GC
    slots = ((idx_a, rows_a, sem_a), (idx_b, rows_b, sem_b))

    pltpu.sync_copy(idx_hbm.at[pl.ds(base, _GC)], idx_a)
    pltpu.async_copy(tab_hbm.at[idx_a], rows_a, sem_a)

    def body(j, carry):
        for b in (0, 1):
            i = 2 * j + b
            idx_c, rows_c, sem_c = slots[b]
            idx_n, rows_n, sem_n = slots[1 - b]

            @pl.when(i + 1 < nch)
            def _():
                eb_n = base + (i + 1) * _GC
                pltpu.sync_copy(idx_hbm.at[pl.ds(eb_n, _GC)], idx_n)
                pltpu.async_copy(tab_hbm.at[idx_n], rows_n, sem_n)

            pltpu.make_async_copy(tab_hbm.at[idx_c], rows_c, sem_c).wait()
            pltpu.sync_copy(rows_c, out_hbm.at[pl.ds(base + i * _GC, _GC)])
        return carry

    lax.fori_loop(0, nch // 2, body, 0)


@functools.lru_cache(maxsize=None)
def _make_sc_gather(n_edges):
    mesh = plsc.VectorSubcoreMesh(core_axis_name="c", subcore_axis_name="s")
    return pl.kernel(
        functools.partial(_sc_gather_body, n_edges),
        out_type=jax.ShapeDtypeStruct((n_edges, DPAD), _F32),
        mesh=mesh,
        scratch_types=[
            pltpu.VMEM((_GC,), jnp.int32),
            pltpu.VMEM((_GC,), jnp.int32),
            pltpu.VMEM((_GC, DPAD), _F32),
            pltpu.VMEM((_GC, DPAD), _F32),
            pltpu.SemaphoreType.DMA,
            pltpu.SemaphoreType.DMA,
        ],
    )


def _sc_gather(table, idx_flat):
    return _make_sc_gather(idx_flat.shape[0])(table, idx_flat)


def _edgeconv_body(xj_ref, h_ref, th_ref, ph_ref, o_ref):
    xi = h_ref[...]
    thb = th_ref[...].astype(_BF16)
    phb = ph_ref[...].astype(_BF16)
    dn = (((1,), (1,)), ((), ()))
    p = lax.dot_general(xi.astype(_BF16), phb, dn, preferred_element_type=_F32)
    m = None
    for k in range(KNB):
        d = (xj_ref[k] - xi).astype(_BF16)
        mk = lax.dot_general(d, thb, dn, preferred_element_type=_F32)
        m = mk if m is None else jnp.maximum(m, mk)
    out = m + p
    o_ref[...] = jnp.where(out >= 0.0, out, 0.2 * out)


def _edgeconv(xj, h_pad, th_pad, ph_pad, roff, nrows):
    dout_pad = th_pad.shape[0]
    rb = 256
    grid = (nrows // rb,)
    ro = roff // rb
    xj3 = xj.reshape(KNB, nrows, DPAD)
    return pl.pallas_call(
        _edgeconv_body,
        grid=grid,
        in_specs=[
            pl.BlockSpec((KNB, rb, DPAD), lambda i: (0, i, 0)),
            pl.BlockSpec((rb, DPAD), lambda i: (i + ro, 0)),
            pl.BlockSpec((dout_pad, DPAD), lambda i: (0, 0)),
            pl.BlockSpec((dout_pad, DPAD), lambda i: (0, 0)),
        ],
        out_specs=pl.BlockSpec((rb, dout_pad), lambda i: (i, 0)),
        out_shape=jax.ShapeDtypeStruct((nrows, dout_pad), _F32),
    )(xj3, h_pad, th_pad, ph_pad)


def _proj_pool_body(h1_ref, h2_ref, h3_ref, h4_ref, w_ref, b_ref,
                    pmax_ref, pavg_ref):
    hcat = jnp.concatenate(
        [h1_ref[0][:, :64], h2_ref[0][:, :64], h3_ref[0], h4_ref[0]],
        axis=1)
    dn = (((1,), (1,)), ((), ()))
    pr = lax.dot_general(hcat.astype(_BF16), w_ref[...].astype(_BF16), dn,
                         preferred_element_type=_F32)
    pr = pr + b_ref[...]
    pmax_ref[0] = jnp.max(pr, axis=0, keepdims=True)
    pavg_ref[0] = jnp.sum(pr, axis=0, keepdims=True) * (1.0 / N)


def _proj_pool(hs, proj_w, proj_b):
    dproj, dcat = proj_w.shape
    nb = hs[0].shape[0]
    dims = [h.shape[2] for h in hs]
    specs = [pl.BlockSpec((1, N, d), lambda b: (b, 0, 0)) for d in dims]
    return pl.pallas_call(
        _proj_pool_body,
        grid=(nb,),
        in_specs=specs + [
            pl.BlockSpec((dproj, dcat), lambda b: (0, 0)),
            pl.BlockSpec((1, dproj), lambda b: (0, 0)),
        ],
        out_specs=[
            pl.BlockSpec((1, 1, dproj), lambda b: (b, 0, 0)),
            pl.BlockSpec((1, 1, dproj), lambda b: (b, 0, 0)),
        ],
        out_shape=[
            jax.ShapeDtypeStruct((nb, 1, dproj), _F32),
            jax.ShapeDtypeStruct((nb, 1, dproj), _F32),
        ],
    )(*hs, proj_w, proj_b.reshape(1, dproj))


def _mlp_body(pmax_ref, pavg_ref, w0_ref, b0_ref, g0_ref, bb0_ref,
              w1_ref, b1_ref, g1_ref, bb1_ref, wo_ref, bo_ref, out_ref):
    h = jnp.concatenate([pmax_ref[...], pavg_ref[...]], axis=1)
    dn = (((1,), (1,)), ((), ()))

    def block(h, w_ref, b_ref, g_ref, bb_ref):
        h = lax.dot_general(h.astype(_BF16), w_ref[...].astype(_BF16), dn,
                            preferred_element_type=_F32)
        h = h + b_ref[...]
        mean = jnp.sum(h, axis=0, keepdims=True) * (1.0 / B)
        d = h - mean
        var = jnp.sum(d * d, axis=0, keepdims=True) * (1.0 / B)
        h = d / jnp.sqrt(var + 1e-5) * g_ref[...] + bb_ref[...]
        return jnp.where(h >= 0.0, h, 0.2 * h)

    h = block(h, w0_ref, b0_ref, g0_ref, bb0_ref)
    h = block(h, w1_ref, b1_ref, g1_ref, bb1_ref)
    out = lax.dot_general(h.astype(_BF16), wo_ref[...].astype(_BF16), dn,
                          preferred_element_type=_F32)
    out_ref[...] = out + bo_ref[...]


def _mlp_head(pmax, pavg, emb_w_0, emb_b_0, bn_g_0, bn_b_0,
              emb_w_1, emb_b_1, bn_g_1, bn_b_1, out_w, out_b):
    args = [
        pmax, pavg,
        emb_w_0, emb_b_0.reshape(1, -1), bn_g_0.reshape(1, -1),
        bn_b_0.reshape(1, -1),
        emb_w_1, emb_b_1.reshape(1, -1), bn_g_1.reshape(1, -1),
        bn_b_1.reshape(1, -1),
        out_w, out_b.reshape(1, -1),
    ]
    nclass = out_w.shape[0]
    return pl.pallas_call(
        _mlp_body,
        out_shape=jax.ShapeDtypeStruct((B, nclass), _F32),
    )(*args)


def _pad_w(w):
    dout, din = w.shape
    dout_pad = 128 if dout < 128 else dout
    return jnp.pad(w, ((0, dout_pad - dout), (0, DPAD - din)))


@jax.jit
def kernel(x, theta_0, phi_0, theta_1, phi_1, theta_2, phi_2, theta_3, phi_3,
           proj_w, proj_b, emb_w_0, emb_b_0, bn_g_0, bn_b_0,
           emb_w_1, emb_b_1, bn_g_1, bn_b_1, out_w, out_b):
    thetas = [_pad_w(w) for w in (theta_0, theta_1, theta_2, theta_3)]
    phis = [_pad_w(w) for w in (phi_0, phi_1, phi_2, phi_3)]
    h_pad = jnp.pad(x.reshape(BN, 3), ((0, 0), (0, DPAD - 3)))
    hs = []
    hb = B // 2
    for li in range(4):
        h3 = h_pad.reshape(B, N, DPAD)
        idx0 = _knn_topk(h3, 0, hb)
        xj0 = _sc_gather(h_pad, idx0.transpose(2, 0, 1).reshape(-1))
        idx1 = _knn_topk(h3, hb, hb)
        xj1 = _sc_gather(h_pad, idx1.transpose(2, 0, 1).reshape(-1))
        ho0 = _edgeconv(xj0, h_pad, thetas[li], phis[li], 0, hb * N)
        ho1 = _edgeconv(xj1, h_pad, thetas[li], phis[li], hb * N, hb * N)
        h_out = jnp.concatenate([ho0, ho1], axis=0)
        hs.append(h_out.reshape(B, N, -1))
        if li < 3:
            h_pad = h_out[:, :DPAD] if h_out.shape[1] > DPAD else h_out
    pmax, pavg = _proj_pool(hs, proj_w, proj_b)
    pmax = pmax.reshape(B, -1)
    pavg = pavg.reshape(B, -1)
    return _mlp_head(pmax, pavg, emb_w_0, emb_b_0, bn_g_0, bn_b_0,
                     emb_w_1, emb_b_1, bn_g_1, bn_b_1, out_w, out_b)
